# Initial kernel scaffold; baseline (speedup 1.0000x reference)
import jax
import jax.numpy as jnp
from jax.experimental import pallas as pl


def kernel(vector, weights):
    raise NotImplementedError("write your pallas kernel here")


# speedup vs baseline: 18.5812x; 18.5812x over previous
import functools

import numpy as np
import jax
import jax.numpy as jnp
from jax import lax
from jax.experimental import pallas as pl
from jax.experimental.pallas import tpu as pltpu
from jax.experimental.pallas import tpu_sc as plsc

NQ = 8
KW = 3
NL = 2
NC, NS, LN = 2, 16, 16


_NGRIDS = 8
_BASE = np.array([0.0, 2 * np.pi / 3, -2 * np.pi / 3])


def _grid_and_pinv():
    grids, rows = [], []
    for dlt in 0.0125 * np.arange(_NGRIDS):
        th = _BASE + dlt
        g = np.stack(np.meshgrid(th, th, th, indexing="ij"), -1).reshape(27, 3)
        grids.append(g)
        m = np.stack([np.ones(3), np.cos(th), np.sin(th)], 1)
        rows.append(np.einsum('ax,by,cz->abcxyz', m, m, m).reshape(27, 27))
    return (np.concatenate(grids).astype(np.float32),
            np.linalg.pinv(np.concatenate(rows)).astype(np.float32))


_GRID, _PINV = _grid_and_pinv()


def _g1(state, U, w):
    st = state.reshape((2,) * NQ)
    st = jnp.tensordot(U, st, axes=((1,), (w,)))
    st = jnp.moveaxis(st, 0, w)
    return st.reshape(-1)


def _gcx(state, c, t):
    st = state.reshape((2,) * NQ)
    s0 = jnp.take(st, 0, axis=c)
    s1 = jnp.take(st, 1, axis=c)
    t2 = t - 1 if t > c else t
    s1 = jnp.flip(s1, axis=t2)
    st = jnp.stack([s0, s1], axis=c)
    return st.reshape(-1)


def _gry(t):
    t = t.astype(jnp.float32)
    ch = jnp.cos(t / 2).astype(jnp.complex64)
    sh = jnp.sin(t / 2).astype(jnp.complex64)
    return jnp.stack([jnp.stack([ch, -sh]), jnp.stack([sh, ch])])


def _grz(t):
    tc = t.astype(jnp.complex64)
    e0 = jnp.exp(-0.5j * tc)
    e1 = jnp.exp(0.5j * tc)
    z = jnp.zeros((), jnp.complex64)
    return jnp.stack([jnp.stack([e0, z]), jnp.stack([z, e1])])


def _grot(state, phi, theta, omega, w):
    state = _g1(state, _grz(phi), w)
    state = _g1(state, _gry(theta), w)
    state = _g1(state, _grz(omega), w)
    return state


def _glayer(state, wl):
    for i in range(NQ):
        state = _grot(state, wl[i, 0], wl[i, 1], wl[i, 2], i)
    for i in range(NQ):
        state = _gcx(state, i, (i + 1) % NQ)
    return state


def _gcircuit(inputs, weights):
    state = jnp.zeros((2 ** NQ,), jnp.complex64).at[0].set(1.0 + 0.0j)
    state = _glayer(state, weights[0])
    for inp in range(KW):
        state = _g1(state, _gry(inputs[inp]), inp)
    for i in range(1, NL + 1):
        state = _glayer(state, weights[i])
    probs = state.real ** 2 + state.imag ** 2
    p = probs.reshape((2,) * NQ)
    z = [jnp.take(p, 0, axis=j).sum() - jnp.take(p, 1, axis=j).sum()
         for j in range(NQ)]
    return jnp.stack(z)


def _weights_to_coeffs(weights):
    z = jax.vmap(lambda x: _gcircuit(x, weights))(jnp.asarray(_GRID))
    W = jnp.einsum('ap,pj->aj', jnp.asarray(_PINV), z,
                   precision=jax.lax.Precision.HIGHEST)
    return W.T


_TWO_OVER_PI = 0.6366197723675814
_RND = 12582912.0
_P1, _P2, _P3 = 1.5703125, 4.837512969970703e-4, 7.549789948768648e-8
_S1, _S2, _S3 = -1.6666654611e-1, 8.3321608736e-3, -1.9515295891e-4
_C1, _C2, _C3 = 4.166664568298827e-2, -1.388731625493765e-3, 2.443315711809948e-5


def _sincos16(x):
    n = (x * _TWO_OVER_PI + _RND) - _RND
    q = n.astype(jnp.int32)
    r = x - n * _P1
    r = r - n * _P2
    r = r - n * _P3
    r2 = r * r
    sp = r + r * r2 * (_S1 + r2 * (_S2 + r2 * _S3))
    cp = 1.0 - 0.5 * r2 + r2 * r2 * (_C1 + r2 * (_C2 + r2 * _C3))
    swap = (q & 1) == 1
    sb = jnp.where(swap, cp, sp)
    cb = jnp.where(swap, sp, cp)
    s = jnp.where((q & 2) == 2, -sb, sb)
    c = jnp.where(((q + 1) & 2) == 2, -cb, cb)
    return c, s


def _quanv_body(v_hbm, w_hbm, out_hbm, x_v, cpad, spad, w_v, o_v):
    wid = lax.axis_index("c") * NS + lax.axis_index("s")
    pltpu.sync_copy(v_hbm.at[wid], x_v)
    pltpu.sync_copy(w_hbm, w_v)

    ones = jnp.full((LN,), 1.0, jnp.float32)
    zeros = jnp.zeros((LN,), jnp.float32)

    cpad[pl.ds(0, LN)] = ones
    spad[pl.ds(0, LN)] = zeros
    cpad[pl.ds(272, LN)] = ones
    spad[pl.ds(272, LN)] = zeros

    for k in range(16):
        c, s = _sincos16(x_v[pl.ds(LN * k, LN)])
        cpad[pl.ds(LN + LN * k, LN)] = c
        spad[pl.ds(LN + LN * k, LN)] = s

    wvecs = [w_v[pl.ds(LN * i, LN)] for i in range(224 // LN)]
    w = [wvecs[i // LN][i % LN] for i in range(NQ * 27)]

    macc = [zeros] * NQ
    for k in range(16):
        base = LN + LN * k
        c0 = cpad[pl.ds(base - 1, LN)]
        s0 = spad[pl.ds(base - 1, LN)]
        c1 = cpad[pl.ds(base, LN)]
        s1 = spad[pl.ds(base, LN)]
        c2 = cpad[pl.ds(base + 1, LN)]
        s2 = spad[pl.ds(base + 1, LN)]
        f0 = (None, c0, s0)
        f1 = (None, c1, s1)
        for j in range(NQ):
            accj = None
            for a in range(3):
                t1 = None
                for b in range(3):
                    o = j * 27 + a * 9 + b * 3
                    t2 = w[o] + w[o + 1] * c2 + w[o + 2] * s2
                    if b:
                        t2 = t2 * f1[b]
                    t1 = t2 if t1 is None else t1 + t2
                if a:
                    t1 = t1 * f0[a]
                accj = t1 if accj is None else accj + t1
            macc[j] = jnp.maximum(macc[j], accj)

    dn = lax.GatherDimensionNumbers(
        offset_dims=(), collapsed_slice_dims=(0,), start_index_map=(0,))
    lane = lax.iota(jnp.int32, LN)
    row = zeros
    for j in range(NQ):
        m = macc[j]
        for st in (1, 2, 4, 8):
            idx = (lane ^ st).reshape(LN, 1)
            m = jnp.maximum(m, lax.gather(
                m, idx, dn, slice_sizes=(1,),
                mode=lax.GatherScatterMode.PROMISE_IN_BOUNDS))
        row = jnp.where(lane == j, m, row)
    o_v[...] = row
    pltpu.sync_copy(o_v, out_hbm.at[wid])


@functools.cache
def _quanv_sc():
    return functools.partial(
        pl.kernel,
        out_type=jax.ShapeDtypeStruct((NC * NS, LN), jnp.float32),
        mesh=plsc.VectorSubcoreMesh(
            core_axis_name="c", subcore_axis_name="s",
            num_cores=NC, num_subcores=NS,
        ),
        scratch_types=[
            pltpu.VMEM((256,), jnp.float32),
            pltpu.VMEM((288,), jnp.float32),
            pltpu.VMEM((288,), jnp.float32),
            pltpu.VMEM((224,), jnp.float32),
            pltpu.VMEM((LN,), jnp.float32),
        ],
    )(_quanv_body)


def kernel(vector, weights):
    bs, ch, l = vector.shape
    v = vector.mean(axis=-2) if ch > 1 else vector[:, 0, :]
    coeffs = _weights_to_coeffs(weights)
    wflat = jnp.zeros((224,), jnp.float32).at[:NQ * 27].set(coeffs.reshape(-1))
    out = _quanv_sc()(v, wflat)
    return out[:, :NQ, None]
